# baseline (speedup 1.0000x reference)
import functools

import jax
import jax.numpy as jnp
import numpy as np
from jax import lax
from jax.experimental import pallas as pl
from jax.experimental.pallas import tpu as pltpu

_M = 16
_K = 3
_D = 512
_EXCL = 4
_T = 256
_BPS = 4
_NEG = np.float32(-np.inf)


def _better(av, af, bv, bf):
    return (av > bv) | ((av == bv) & (af < bf))


def _mp_kernel(ts_ref, x_ref, w_ref, b_ref, out_ref,
               wzf_ref, wz3_ref, masks_ref, tmax_ref, *, n, s_len, bps):
    nt = s_len // _T
    lastc0 = (nt - 1) * _T

    r_io = lax.broadcasted_iota(jnp.int32, (_T, _T), 0)
    c_io = lax.broadcasted_iota(jnp.int32, (_T, _T), 1)
    zero = jnp.zeros((_T, _T), jnp.float32)
    diag_m = jnp.where(c_io - r_io > _EXCL, 0.0, _NEG)
    sup_m = jnp.where(c_io + _T - r_io > _EXCL, 0.0, _NEG)
    last_m = jnp.where(c_io < n - lastc0, 0.0, _NEG)
    masks_ref[0] = zero
    masks_ref[1] = diag_m
    masks_ref[2] = sup_m
    masks_ref[3] = last_m
    masks_ref[4] = diag_m + last_m
    masks_ref[5] = sup_m + last_m
    class_mask = [zero, diag_m, sup_m, last_m, diag_m + last_m, sup_m + last_m]
    io64 = lax.broadcasted_iota(jnp.int32, (nt * nt, 1), 0)
    tmax_ref[...] = jnp.full((bps, nt * nt, 128), -np.inf, jnp.float32)

    for bi in range(bps):
        _one_batch(bi, ts_ref, x_ref, w_ref, b_ref, out_ref,
                   wzf_ref.at[bi], wz3_ref.at[bi], masks_ref, tmax_ref.at[bi],
                   class_mask, r_io, c_io, io64, n=n, s_len=s_len)


def _one_batch(bi, ts_ref, x_ref, w_ref, b_ref, out_ref,
               wzf_ref, wz3_ref, masks_ref, tmax_ref,
               class_mask, r_io, c_io, io64, *, n, s_len):
    nt = s_len // _T

    w = jnp.stack([ts_ref[bi, 0, pl.ds(k, s_len)] for k in range(_M)], axis=0)
    mu = jnp.mean(w, axis=0, keepdims=True)
    sd = jnp.sqrt(jnp.mean((w - mu) ** 2, axis=0, keepdims=True)) + 1e-8
    wz = (w - mu) / sd
    wzf_ref[...] = wz
    for c in range(nt):
        wz3_ref[c] = wz[:, c * _T:(c + 1) * _T]

    def tile_class(rt, ct):
        if ct == rt:
            return 4 if ct == nt - 1 else 1
        if ct == rt + 1:
            return 5 if ct == nt - 1 else 2
        if ct == nt - 1:
            return 3
        return 0

    for rt in range(nt):
        wd = (nt - rt) * _T
        d = lax.dot_general(wzf_ref[:, pl.ds(rt * _T, _T)],
                            wzf_ref[:, pl.ds(rt * _T, wd)],
                            (((0,), (0,)), ((), ())),
                            preferred_element_type=jnp.float32)
        for ct in range(rt, nt):
            md = tile_class(rt, ct)
            tv = d[:, (ct - rt) * _T:(ct - rt + 1) * _T]
            if md != 0:
                tv = tv + class_mask[md]
            m = jnp.max(tv, axis=0)
            tmax_ref[rt * nt + ct] = jnp.maximum(m[:128], m[128:])

    mt = jnp.max(tmax_ref[...], axis=1, keepdims=True)
    big = np.int32(2**31 - 1)
    ma = jnp.max(mt)
    ta = jnp.min(jnp.where(mt == ma, io64, big))
    mt2 = jnp.where(io64 == ta, _NEG, mt)
    mb = jnp.max(mt2)
    tb = jnp.min(jnp.where(mt2 == mb, io64, big))

    def tile_ids(t):
        rt = jnp.right_shift(t, 3)
        ct = jnp.bitwise_and(t, nt - 1)
        diag = ct == rt
        sup = ct == rt + 1
        last = ct == nt - 1
        md = jnp.where(
            diag, jnp.where(last, 4, 1),
            jnp.where(sup, jnp.where(last, 5, 2), jnp.where(last, 3, 0)))
        return rt, ct, md

    rta, cta, mda = tile_ids(ta)
    rtb, ctb, mdb = tile_ids(tb)

    def tile_score(rt, ct, md):
        d = lax.dot_general(wz3_ref[rt], wz3_ref[ct],
                            (((0,), (0,)), ((), ())),
                            preferred_element_type=jnp.float32)
        score = d + masks_ref[md]
        rows = rt * _T + r_io
        cols = ct * _T + c_io
        flat2 = jnp.left_shift(rows, 11) | cols
        return score, flat2

    score_a, flat_a = tile_score(rta, cta, mda)
    pv1 = jnp.max(score_a)
    pf1 = jnp.min(jnp.where(score_a == pv1, flat_a, big))
    score_a2 = jnp.where(flat_a == pf1, _NEG, score_a)
    pv2 = jnp.max(score_a2)
    pf2 = jnp.min(jnp.where(score_a2 == pv2, flat_a, big))

    score_b, flat_b = tile_score(rtb, ctb, mdb)
    pv3 = jnp.max(score_b)
    pf3 = jnp.min(jnp.where(score_b == pv3, flat_b, big))

    use2 = _better(pv2, pf2, pv3, pf3)
    f0 = pf1
    f1 = jnp.where(use2, pf2, pf3)

    i1 = jnp.right_shift(f0, 11)
    j1 = jnp.bitwise_and(f0, 2047)
    j2 = jnp.bitwise_and(f1, 2047)

    for kk, cc in enumerate((j1, i1, j2)):
        st = jnp.clip(cc - _M // 2, 0, s_len - _M)
        patch = x_ref[bi, pl.ds(st, _M), :]
        ok = lax.dot_general(patch, w_ref[...], (((0,), (1,)), ((), ())),
                             preferred_element_type=jnp.float32)
        out_ref[bi, kk, :, :] = ok + b_ref[0, :][None, :]


def kernel(x, W, b):
    B, S, C = x.shape
    n = S - _M + 1
    nt = S // _T
    ts_pad = jnp.pad(x[:, :, 0], ((0, 0), (0, 128))).reshape(B, 1, S + 128)
    bias2d = b.reshape(1, _D)
    bps = _BPS
    out = pl.pallas_call(
        functools.partial(_mp_kernel, n=n, s_len=S, bps=bps),
        grid=(B // bps,),
        in_specs=[
            pl.BlockSpec((bps, 1, S + 128), lambda bb: (bb, 0, 0)),
            pl.BlockSpec((bps, S, C), lambda bb: (bb, 0, 0)),
            pl.BlockSpec((_D, _M), lambda bb: (0, 0)),
            pl.BlockSpec((1, _D), lambda bb: (0, 0)),
        ],
        out_specs=pl.BlockSpec((bps, _K, C, _D), lambda bb: (bb, 0, 0, 0)),
        out_shape=jax.ShapeDtypeStruct((B, _K, C, _D), jnp.float32),
        scratch_shapes=[
            pltpu.VMEM((bps, _M, S), jnp.float32),
            pltpu.VMEM((bps, nt, _M, _T), jnp.float32),
            pltpu.VMEM((6, _T, _T), jnp.float32),
            pltpu.VMEM((bps, nt * nt, 128), jnp.float32),
        ],
        compiler_params=pltpu.CompilerParams(
            dimension_semantics=("parallel",)),
    )(ts_pad, x, W, bias2d)
    return jnp.transpose(out, (0, 2, 1, 3))


# speedup vs baseline: 1.0013x; 1.0013x over previous
import functools

import jax
import jax.numpy as jnp
import numpy as np
from jax import lax
from jax.experimental import pallas as pl
from jax.experimental.pallas import tpu as pltpu

_M = 16
_K = 3
_D = 512
_EXCL = 4
_T = 256
_BPS = 4
_NEG = np.float32(-np.inf)


def _better(av, af, bv, bf):
    return (av > bv) | ((av == bv) & (af < bf))


def _mp_kernel(ts_ref, x_ref, w_ref, b_ref, out_ref,
               wzf_ref, wz3_ref, masks_ref, tmax_ref, *, n, s_len, bps):
    nt = s_len // _T
    lastc0 = (nt - 1) * _T

    r_io = lax.broadcasted_iota(jnp.int32, (_T, _T), 0)
    c_io = lax.broadcasted_iota(jnp.int32, (_T, _T), 1)
    zero = jnp.zeros((_T, _T), jnp.float32)
    diag_m = jnp.where(c_io - r_io > _EXCL, 0.0, _NEG)
    sup_m = jnp.where(c_io + _T - r_io > _EXCL, 0.0, _NEG)
    last_m = jnp.where(c_io < n - lastc0, 0.0, _NEG)
    masks_ref[0] = zero
    masks_ref[1] = diag_m
    masks_ref[2] = sup_m
    masks_ref[3] = last_m
    masks_ref[4] = diag_m + last_m
    masks_ref[5] = sup_m + last_m
    class_mask = [zero, diag_m, sup_m, last_m, diag_m + last_m, sup_m + last_m]
    io64 = lax.broadcasted_iota(jnp.int32, (nt * nt, 1), 0)
    tmax_ref[...] = jnp.full((bps, nt * nt, 128), -np.inf, jnp.float32)

    for bi in range(bps):
        _one_batch(bi, ts_ref, x_ref, w_ref, b_ref, out_ref,
                   wzf_ref.at[bi], wz3_ref.at[bi], masks_ref, tmax_ref.at[bi],
                   class_mask, r_io, c_io, io64, n=n, s_len=s_len)


def _one_batch(bi, ts_ref, x_ref, w_ref, b_ref, out_ref,
               wzf_ref, wz3_ref, masks_ref, tmax_ref,
               class_mask, r_io, c_io, io64, *, n, s_len):
    nt = s_len // _T

    w = jnp.stack([ts_ref[bi, 0, pl.ds(k, s_len)] for k in range(_M)], axis=0)
    mu = jnp.mean(w, axis=0, keepdims=True)
    sd = jnp.sqrt(jnp.mean((w - mu) ** 2, axis=0, keepdims=True)) + 1e-8
    wz = (w - mu) / sd
    wzf_ref[...] = wz
    for c in range(nt):
        wz3_ref[c] = wz[:, c * _T:(c + 1) * _T]

    def tile_class(rt, ct):
        if ct == rt:
            return 4 if ct == nt - 1 else 1
        if ct == rt + 1:
            return 5 if ct == nt - 1 else 2
        if ct == nt - 1:
            return 3
        return 0

    def strip_mm(rt):
        wd = (nt - rt) * _T
        return lax.dot_general(wzf_ref[:, pl.ds(rt * _T, _T)],
                               wzf_ref[:, pl.ds(rt * _T, wd)],
                               (((0,), (0,)), ((), ())),
                               preferred_element_type=jnp.float32)

    def strip_reduce(rt, d):
        for ct in range(rt, nt):
            md = tile_class(rt, ct)
            tv = d[:, (ct - rt) * _T:(ct - rt + 1) * _T]
            if md != 0:
                tv = tv + class_mask[md]
            m = jnp.max(tv, axis=0)
            tmax_ref[rt * nt + ct] = jnp.maximum(m[:128], m[128:])

    d_prev = strip_mm(0)
    for rt in range(1, nt):
        d_cur = strip_mm(rt)
        strip_reduce(rt - 1, d_prev)
        d_prev = d_cur
    strip_reduce(nt - 1, d_prev)

    mt = jnp.max(tmax_ref[...], axis=1, keepdims=True)
    big = np.int32(2**31 - 1)
    ma = jnp.max(mt)
    ta = jnp.min(jnp.where(mt == ma, io64, big))
    mt2 = jnp.where(io64 == ta, _NEG, mt)
    mb = jnp.max(mt2)
    tb = jnp.min(jnp.where(mt2 == mb, io64, big))

    def tile_ids(t):
        rt = jnp.right_shift(t, 3)
        ct = jnp.bitwise_and(t, nt - 1)
        diag = ct == rt
        sup = ct == rt + 1
        last = ct == nt - 1
        md = jnp.where(
            diag, jnp.where(last, 4, 1),
            jnp.where(sup, jnp.where(last, 5, 2), jnp.where(last, 3, 0)))
        return rt, ct, md

    rta, cta, mda = tile_ids(ta)
    rtb, ctb, mdb = tile_ids(tb)

    def tile_score(rt, ct, md):
        d = lax.dot_general(wz3_ref[rt], wz3_ref[ct],
                            (((0,), (0,)), ((), ())),
                            preferred_element_type=jnp.float32)
        score = d + masks_ref[md]
        rows = rt * _T + r_io
        cols = ct * _T + c_io
        flat2 = jnp.left_shift(rows, 11) | cols
        return score, flat2

    score_a, flat_a = tile_score(rta, cta, mda)
    pv1 = jnp.max(score_a)
    pf1 = jnp.min(jnp.where(score_a == pv1, flat_a, big))
    score_a2 = jnp.where(flat_a == pf1, _NEG, score_a)
    pv2 = jnp.max(score_a2)
    pf2 = jnp.min(jnp.where(score_a2 == pv2, flat_a, big))

    score_b, flat_b = tile_score(rtb, ctb, mdb)
    pv3 = jnp.max(score_b)
    pf3 = jnp.min(jnp.where(score_b == pv3, flat_b, big))

    use2 = _better(pv2, pf2, pv3, pf3)
    f0 = pf1
    f1 = jnp.where(use2, pf2, pf3)

    i1 = jnp.right_shift(f0, 11)
    j1 = jnp.bitwise_and(f0, 2047)
    j2 = jnp.bitwise_and(f1, 2047)

    for kk, cc in enumerate((j1, i1, j2)):
        st = jnp.clip(cc - _M // 2, 0, s_len - _M)
        patch = x_ref[bi, pl.ds(st, _M), :]
        ok = lax.dot_general(patch, w_ref[...], (((0,), (1,)), ((), ())),
                             preferred_element_type=jnp.float32)
        out_ref[bi, kk, :, :] = ok + b_ref[0, :][None, :]


def kernel(x, W, b):
    B, S, C = x.shape
    n = S - _M + 1
    nt = S // _T
    ts_pad = jnp.pad(x[:, :, 0], ((0, 0), (0, 128))).reshape(B, 1, S + 128)
    bias2d = b.reshape(1, _D)
    bps = _BPS
    out = pl.pallas_call(
        functools.partial(_mp_kernel, n=n, s_len=S, bps=bps),
        grid=(B // bps,),
        in_specs=[
            pl.BlockSpec((bps, 1, S + 128), lambda bb: (bb, 0, 0)),
            pl.BlockSpec((bps, S, C), lambda bb: (bb, 0, 0)),
            pl.BlockSpec((_D, _M), lambda bb: (0, 0)),
            pl.BlockSpec((1, _D), lambda bb: (0, 0)),
        ],
        out_specs=pl.BlockSpec((bps, _K, C, _D), lambda bb: (bb, 0, 0, 0)),
        out_shape=jax.ShapeDtypeStruct((B, _K, C, _D), jnp.float32),
        scratch_shapes=[
            pltpu.VMEM((bps, _M, S), jnp.float32),
            pltpu.VMEM((bps, nt, _M, _T), jnp.float32),
            pltpu.VMEM((6, _T, _T), jnp.float32),
            pltpu.VMEM((bps, nt * nt, 128), jnp.float32),
        ],
        compiler_params=pltpu.CompilerParams(
            dimension_semantics=("parallel",)),
    )(ts_pad, x, W, bias2d)
    return jnp.transpose(out, (0, 2, 1, 3))
